# baseline (speedup 1.0000x reference)
import functools

import jax
import jax.numpy as jnp
from jax import lax
from jax.experimental import pallas as pl
from jax.experimental.pallas import tpu as pltpu
from jax.experimental.pallas import tpu_sc as plsc

NC = 2
NS = 16
NW = NC * NS
LANES = 16


def _make_transpose_kernel(V, D):
    TCOL = 128
    n_full = V // TCOL
    rem = V - n_full * TCOL
    per = n_full // NW
    extra = n_full - per * NW
    n_j = D // LANES

    mesh = plsc.VectorSubcoreMesh(
        core_axis_name="c", subcore_axis_name="s", num_cores=NC,
        num_subcores=NS)

    @functools.partial(
        pl.kernel,
        mesh=mesh,
        compiler_params=pltpu.CompilerParams(
            use_tc_tiling_on_sc=True, needs_layout_passes=False),
        out_type=jax.ShapeDtypeStruct((V * D,), jnp.float32),
        scratch_types=[
            pltpu.VMEM((D, TCOL), jnp.float32),
            pltpu.VMEM((D, TCOL), jnp.float32),
            pltpu.VMEM((D * 136,), jnp.float32),
            pltpu.VMEM((TCOL * D,), jnp.float32),
            pltpu.VMEM((TCOL * D,), jnp.float32),
            pltpu.SemaphoreType.DMA,
            pltpu.SemaphoreType.DMA,
            pltpu.SemaphoreType.DMA,
        ],
    )
    def transpose_k(tT_hbm, tail_hbm, out_hbm, stage0, stage1, skew, outb0,
                    outb1, semi0, semi1, semo):
        wid = lax.axis_index("s") * NC + lax.axis_index("c")
        cnt = per + jnp.where(wid < extra, 1, 0)
        start = per * wid + jnp.minimum(wid, extra)

        iota = lax.iota(jnp.int32, 16)
        skew_base = [(iota + LANES * j) * 136 for j in range(n_j)]

        def fire(k, stage, sem):
            col0 = (start + k) * TCOL
            pltpu.async_copy(
                tT_hbm.at[:, pl.ds(col0, TCOL)], stage, sem)

        def drain(k, stage, sem):
            col0 = (start + k) * TCOL
            pltpu.make_async_copy(
                tT_hbm.at[:, pl.ds(col0, TCOL)], stage, sem).wait()

        def transpose_block(stage, outb):
            @plsc.parallel_loop(0, D, unroll=4)
            def dbody(d):
                for g in range(TCOL // LANES):
                    skew[pl.ds(d * 136 + LANES * g, LANES)] = (
                        stage[d, pl.ds(LANES * g, LANES)])

            @plsc.parallel_loop(0, TCOL, unroll=8)
            def cbody(c):
                base = c * D
                for j in range(n_j):
                    v = plsc.load_gather(skew, [skew_base[j] + c])
                    outb[pl.ds(base + LANES * j, LANES)] = v

        fire(0, stage0, semi0)

        def body2(i, _):
            k = i * 2

            @pl.when(k < cnt)
            def _():
                drain(k, stage0, semi0)

                @pl.when(k + 1 < cnt)
                def _():
                    fire(k + 1, stage1, semi1)

                transpose_block(stage0, outb0)
                col0 = (start + k) * TCOL
                pltpu.sync_copy(outb0, out_hbm.at[pl.ds(col0 * D, TCOL * D)])

            @pl.when(k + 1 < cnt)
            def _():
                drain(k + 1, stage1, semi1)

                @pl.when(k + 2 < cnt)
                def _():
                    fire(k + 2, stage0, semi0)

                transpose_block(stage1, outb1)
                col1 = (start + k + 1) * TCOL
                pltpu.sync_copy(outb1, out_hbm.at[pl.ds(col1 * D, TCOL * D)])
            return 0

        lax.fori_loop(0, (per + 2) // 2, body2, 0)

        if rem:
            @pl.when(wid == NW - 1)
            def _():
                n = rem * D
                pltpu.sync_copy(tail_hbm, outb0.at[pl.ds(0, n)])
                pltpu.sync_copy(outb0.at[pl.ds(0, n)],
                                out_hbm.at[pl.ds(n_full * TCOL * D, n)])

    return transpose_k


def _make_pool_kernel(B, L, V, D):
    assert B % NW == 0
    b_per_w = B // NW
    chunks = []
    off = 0
    while off < L:
        n = min(128, L - off)
        chunks.append((off, n))
        off += n
    n_j = D // LANES
    inv_l = 1.0 / float(L)

    mesh = plsc.VectorSubcoreMesh(
        core_axis_name="c", subcore_axis_name="s", num_cores=NC,
        num_subcores=NS)

    @functools.partial(
        pl.kernel,
        mesh=mesh,
        compiler_params=pltpu.CompilerParams(use_tc_tiling_on_sc=False),
        out_type=jax.ShapeDtypeStruct((B, D), jnp.float32),
        scratch_types=[
            pltpu.VMEM((b_per_w, L), jnp.int32),
            pltpu.VMEM((L, D), jnp.float32),
            pltpu.VMEM((L, D), jnp.float32),
            pltpu.VMEM((b_per_w, D), jnp.float32),
            pltpu.SemaphoreType.DMA,
            pltpu.SemaphoreType.DMA,
        ],
    )
    def pool(x_hbm, table_hbm, out_hbm, idx_v, buf0, buf1, pooled_v,
             sem0, sem1):
        wid = lax.axis_index("s") * NC + lax.axis_index("c")
        base = wid * b_per_w

        pltpu.sync_copy(x_hbm.at[pl.ds(base, b_per_w)], idx_v)

        def fire(r, buf, sem):
            for (o, n) in chunks:
                pltpu.async_copy(
                    table_hbm.at[idx_v.at[r, pl.ds(o, n)]],
                    buf.at[pl.ds(o, n)], sem)

        def drain(r, buf, sem):
            for (o, n) in chunks:
                pltpu.make_async_copy(
                    table_hbm.at[idx_v.at[r, pl.ds(o, n)]],
                    buf.at[pl.ds(o, n)], sem).wait()

        half = L // 2
        lodd = L - half

        def accum(r, buf):
            zero = tuple(jnp.zeros((LANES,), jnp.float32)
                         for _ in range(2 * n_j))

            def body(s, accs):
                out = [a + buf[s, pl.ds(j * LANES, LANES)]
                       for j, a in enumerate(accs[:n_j])]
                out += [a + buf[half + s, pl.ds(j * LANES, LANES)]
                        for j, a in enumerate(accs[n_j:])]
                return tuple(out)
            accs = plsc.parallel_loop(0, half, unroll=4, carry=zero)(body)
            accs = list(accs)
            if lodd != half:
                for j in range(n_j):
                    accs[j] = accs[j] + buf[L - 1, pl.ds(j * LANES, LANES)]
            for j in range(n_j):
                pooled_v[r, pl.ds(j * LANES, LANES)] = (
                    (accs[j] + accs[n_j + j]) * inv_l)

        fire(0, buf0, sem0)

        def body2(i, _):
            r = i * 2
            drain(r, buf0, sem0)
            fire(r + 1, buf1, sem1)
            accum(r, buf0)
            drain(r + 1, buf1, sem1)

            @pl.when(r + 2 < b_per_w)
            def _():
                fire(r + 2, buf0, sem0)

            accum(r + 1, buf1)
            return 0

        lax.fori_loop(0, b_per_w // 2, body2, 0)

        pltpu.sync_copy(pooled_v, out_hbm.at[pl.ds(base, b_per_w)])

    return pool


def _mm_body(p_ref, w_ref, b_ref, o_ref):
    o_ref[...] = lax.dot_general(
        p_ref[...], w_ref[...],
        dimension_numbers=(((1,), (1,)), ((), ())),
        preferred_element_type=jnp.float32) + b_ref[...]


def kernel(x, table, W, b):
    B, L = x.shape
    V, D = table.shape
    C = W.shape[0]

    rem = V % 128
    tail = table[V - rem:, :].reshape(rem * D)
    table_lin = _make_transpose_kernel(V, D)(table.T, tail)
    pooled = _make_pool_kernel(B, L, V, D)(
        x.astype(jnp.int32), table_lin.reshape(V, D))

    logit = pl.pallas_call(
        _mm_body,
        out_shape=jax.ShapeDtypeStruct((B, C), jnp.float32),
    )(pooled, W, b.reshape(1, C))
    return logit


# speedup vs baseline: 1.1291x; 1.1291x over previous
import functools

import jax
import jax.numpy as jnp
from jax import lax
from jax.experimental import pallas as pl
from jax.experimental.pallas import tpu as pltpu
from jax.experimental.pallas import tpu_sc as plsc

NC = 2
NS = 16
NW = NC * NS
LANES = 16


def _make_transpose_kernel(V, D):
    TCOL = 128
    n_full = V // TCOL
    rem = V - n_full * TCOL
    per = n_full // NW
    extra = n_full - per * NW
    n_j = D // LANES

    mesh = plsc.VectorSubcoreMesh(
        core_axis_name="c", subcore_axis_name="s", num_cores=NC,
        num_subcores=NS)

    DW = D // 2

    @functools.partial(
        pl.kernel,
        mesh=mesh,
        compiler_params=pltpu.CompilerParams(
            use_tc_tiling_on_sc=True, needs_layout_passes=False),
        out_type=jax.ShapeDtypeStruct((V * DW,), jnp.int32),
        scratch_types=[
            pltpu.VMEM((D, TCOL), jnp.float32),
            pltpu.VMEM((D, TCOL), jnp.float32),
            pltpu.VMEM((DW * 136,), jnp.int32),
            pltpu.VMEM((TCOL * DW,), jnp.int32),
            pltpu.VMEM((TCOL * DW,), jnp.int32),
            pltpu.SemaphoreType.DMA,
            pltpu.SemaphoreType.DMA,
            pltpu.SemaphoreType.DMA,
        ],
    )
    def transpose_k(tT_hbm, tail_hbm, out_hbm, stage0, stage1, skew, outb0,
                    outb1, semi0, semi1, semo):
        wid = lax.axis_index("s") * NC + lax.axis_index("c")
        cnt = per + jnp.where(wid < extra, 1, 0)
        start = per * wid + jnp.minimum(wid, extra)

        iota = lax.iota(jnp.int32, 16)
        skew_base = [(iota + LANES * j) * 136 for j in range(DW // LANES)]

        def fire(k, stage, sem):
            col0 = (start + k) * TCOL
            pltpu.async_copy(
                tT_hbm.at[:, pl.ds(col0, TCOL)], stage, sem)

        def drain(k, stage, sem):
            col0 = (start + k) * TCOL
            pltpu.make_async_copy(
                tT_hbm.at[:, pl.ds(col0, TCOL)], stage, sem).wait()

        def transpose_block(stage, outb):
            @plsc.parallel_loop(0, DW, unroll=4)
            def dbody(dw):
                for g in range(TCOL // LANES):
                    a = stage[2 * dw, pl.ds(LANES * g, LANES)]
                    b = stage[2 * dw + 1, pl.ds(LANES * g, LANES)]
                    p = plsc.pack(a, b, format=plsc.PackFormat.INTERLEAVED)
                    skew[pl.ds(dw * 136 + LANES * g, LANES)] = (
                        plsc.bitcast(p, jnp.int32))

            @plsc.parallel_loop(0, TCOL, unroll=8)
            def cbody(c):
                base = c * DW
                for j in range(DW // LANES):
                    v = plsc.load_gather(skew, [skew_base[j] + c])
                    outb[pl.ds(base + LANES * j, LANES)] = v

        fire(0, stage0, semi0)

        def body2(i, _):
            k = i * 2

            @pl.when(k < cnt)
            def _():
                drain(k, stage0, semi0)

                @pl.when(k + 1 < cnt)
                def _():
                    fire(k + 1, stage1, semi1)

                transpose_block(stage0, outb0)
                col0 = (start + k) * TCOL
                pltpu.sync_copy(outb0, out_hbm.at[pl.ds(col0 * DW, TCOL * DW)])

            @pl.when(k + 1 < cnt)
            def _():
                drain(k + 1, stage1, semi1)

                @pl.when(k + 2 < cnt)
                def _():
                    fire(k + 2, stage0, semi0)

                transpose_block(stage1, outb1)
                col1 = (start + k + 1) * TCOL
                pltpu.sync_copy(outb1, out_hbm.at[pl.ds(col1 * DW, TCOL * DW)])
            return 0

        lax.fori_loop(0, (per + 2) // 2, body2, 0)

        if rem:
            @pl.when(wid == NW - 1)
            def _():
                n = rem * DW
                pltpu.sync_copy(tail_hbm, outb0.at[pl.ds(0, n)])
                pltpu.sync_copy(outb0.at[pl.ds(0, n)],
                                out_hbm.at[pl.ds(n_full * TCOL * DW, n)])

    return transpose_k


def _make_pool_kernel(B, L, V, D):
    assert B % NW == 0
    b_per_w = B // NW
    chunks = []
    off = 0
    while off < L:
        n = min(128, L - off)
        chunks.append((off, n))
        off += n
    n_j = D // LANES
    DW = D // 2
    n_jw = DW // LANES
    inv_l = 1.0 / float(L)

    mesh = plsc.VectorSubcoreMesh(
        core_axis_name="c", subcore_axis_name="s", num_cores=NC,
        num_subcores=NS)

    @functools.partial(
        pl.kernel,
        mesh=mesh,
        compiler_params=pltpu.CompilerParams(
            use_tc_tiling_on_sc=False, needs_layout_passes=False),
        out_type=jax.ShapeDtypeStruct((B, D), jnp.float32),
        scratch_types=[
            pltpu.VMEM((b_per_w, L), jnp.int32),
            pltpu.VMEM((L, DW), jnp.int32),
            pltpu.VMEM((L, DW), jnp.int32),
            pltpu.VMEM((b_per_w, D), jnp.float32),
            pltpu.SemaphoreType.DMA,
            pltpu.SemaphoreType.DMA,
        ],
    )
    def pool(x_hbm, table_hbm, out_hbm, idx_v, buf0, buf1, pooled_v,
             sem0, sem1):
        wid = lax.axis_index("s") * NC + lax.axis_index("c")
        base = wid * b_per_w

        pltpu.sync_copy(x_hbm.at[pl.ds(base, b_per_w)], idx_v)

        def fire(r, buf, sem):
            for (o, n) in chunks:
                pltpu.async_copy(
                    table_hbm.at[idx_v.at[r, pl.ds(o, n)]],
                    buf.at[pl.ds(o, n)], sem)

        def drain(r, buf, sem):
            for (o, n) in chunks:
                pltpu.make_async_copy(
                    table_hbm.at[idx_v.at[r, pl.ds(o, n)]],
                    buf.at[pl.ds(o, n)], sem).wait()

        def accum(r, buf):
            zero = tuple(jnp.zeros((LANES,), jnp.float32)
                         for _ in range(2 * n_jw))

            def body(s, accs):
                out = list(accs)
                for j in range(n_jw):
                    w = buf[s, pl.ds(j * LANES, LANES)]
                    a, b = plsc.unpack(
                        plsc.bitcast(w, jnp.bfloat16),
                        format=plsc.PackFormat.INTERLEAVED)
                    out[2 * j] = out[2 * j] + a
                    out[2 * j + 1] = out[2 * j + 1] + b
                return tuple(out)
            accs = plsc.parallel_loop(0, L, unroll=4, carry=zero)(body)
            for j in range(2 * n_jw):
                pooled_v[r, pl.ds(j * LANES, LANES)] = accs[j] * inv_l

        fire(0, buf0, sem0)

        def body2(i, _):
            r = i * 2
            drain(r, buf0, sem0)
            fire(r + 1, buf1, sem1)
            accum(r, buf0)
            drain(r + 1, buf1, sem1)

            @pl.when(r + 2 < b_per_w)
            def _():
                fire(r + 2, buf0, sem0)

            accum(r + 1, buf1)
            return 0

        lax.fori_loop(0, b_per_w // 2, body2, 0)

        pltpu.sync_copy(pooled_v, out_hbm.at[pl.ds(base, b_per_w)])

    return pool


def _mm_body(p_ref, w_ref, b_ref, o_ref):
    o_ref[...] = lax.dot_general(
        p_ref[...], w_ref[...],
        dimension_numbers=(((1,), (1,)), ((), ())),
        preferred_element_type=jnp.float32) + b_ref[...]


def kernel(x, table, W, b):
    B, L = x.shape
    V, D = table.shape
    C = W.shape[0]

    rem = V % 128
    DW = D // 2
    tb = table[V - rem:, :].astype(jnp.bfloat16)
    au = lax.bitcast_convert_type(tb[:, 0::2], jnp.uint16).astype(jnp.uint32)
    bu = lax.bitcast_convert_type(tb[:, 1::2], jnp.uint16).astype(jnp.uint32)
    tail = lax.bitcast_convert_type(
        au | (bu << 16), jnp.int32).reshape(rem * DW)

    table_lin = _make_transpose_kernel(V, D)(table.T, tail)
    pooled = _make_pool_kernel(B, L, V, D)(
        x.astype(jnp.int32), table_lin.reshape(V, DW))

    perm = []
    for j in range(D // 32):
        perm += [32 * j + 2 * k for k in range(16)]
        perm += [32 * j + 2 * k + 1 for k in range(16)]
    Wp = W[:, jnp.array(perm, dtype=jnp.int32)]

    logit = pl.pallas_call(
        _mm_body,
        out_shape=jax.ShapeDtypeStruct((B, C), jnp.float32),
    )(pooled, Wp, b.reshape(1, C))
    return logit
